# baseline (speedup 1.0000x reference)
import jax
import jax.numpy as jnp
from jax.experimental import pallas as pl
from jax.experimental.pallas import tpu as pltpu
from jax.experimental.pallas import tpu_sc as plsc

HIDDEN = 128
EPS = 1e-5
GATHER_WINDOW = 256
TC_BLOCK = 1024
AS_CLASSES = 256
POS_CLASSES = 256


def _sc_gather_rows(table, flat_idx):
    n = flat_idx.shape[0]
    idx2 = flat_idx.reshape(1, n)
    mesh = plsc.VectorSubcoreMesh(core_axis_name="c", subcore_axis_name="s")

    @pl.kernel(
        out_type=jax.ShapeDtypeStruct((n, HIDDEN), table.dtype),
        mesh=mesh,
    )
    def gather_kernel(x_hbm, i_hbm, o_hbm):
        def body(i_vmem, o_vmem):
            pltpu.sync_copy(x_hbm.at[i_vmem.at[0]], o_vmem)

        pltpu.emit_pipeline(
            body,
            grid=(n // GATHER_WINDOW,),
            in_specs=[pl.BlockSpec((1, GATHER_WINDOW), index_map=lambda i: (0, i))],
            out_specs=[pl.BlockSpec((GATHER_WINDOW, HIDDEN), index_map=lambda i: (i, 0))],
            core_axis_name=("c", "s"),
            dimension_semantics=(pltpu.PARALLEL,),
        )(i_hbm, o_hbm)

    return gather_kernel(table, idx2)


def _tc_body(w_ref, cidx_ref, ctab_ref, wp_ref, g_ref, b_ref, o_ref):
    t = TC_BLOCK
    cvec = cidx_ref[0, 0, :]
    cb = jnp.broadcast_to(cvec[:, None], (t, AS_CLASSES))

    hi = cb >> 8
    as_oh = (hi == jax.lax.broadcasted_iota(jnp.int32, (t, AS_CLASSES), 1)
             ).astype(jnp.bfloat16)
    a_s = jnp.dot(as_oh, ctab_ref[...], preferred_element_type=jnp.float32)

    lo = cb & 255
    p_oh = (lo == jax.lax.broadcasted_iota(jnp.int32, (t, POS_CLASSES), 1)
            ).astype(jnp.bfloat16)
    p = jnp.dot(p_oh, wp_ref[...], preferred_element_type=jnp.float32)

    x = w_ref[...] + a_s + p

    xb = x.astype(jnp.bfloat16)
    j_mat = jnp.full((HIDDEN, HIDDEN), 1.0 / HIDDEN, dtype=jnp.bfloat16)
    mu = jnp.dot(xb, j_mat, preferred_element_type=jnp.float32)
    m2 = jnp.dot(xb * xb, j_mat, preferred_element_type=jnp.float32)
    var = m2 - mu * mu
    xn = (x - mu) * jax.lax.rsqrt(var + EPS)
    o_ref[...] = xn * g_ref[...] + b_ref[...]


def _tc_sum_ln(w_rows, cidx, C_tab, W_pos_c, gamma, beta):
    n = w_rows.shape[0]
    nb = n // TC_BLOCK
    cidx3 = cidx.reshape(nb, 1, TC_BLOCK)
    return pl.pallas_call(
        _tc_body,
        grid=(nb,),
        in_specs=[
            pl.BlockSpec((TC_BLOCK, HIDDEN), lambda i: (i, 0)),
            pl.BlockSpec((1, 1, TC_BLOCK), lambda i: (i, 0, 0)),
            pl.BlockSpec((AS_CLASSES, HIDDEN), lambda i: (0, 0)),
            pl.BlockSpec((POS_CLASSES, HIDDEN), lambda i: (0, 0)),
            pl.BlockSpec((1, HIDDEN), lambda i: (0, 0)),
            pl.BlockSpec((1, HIDDEN), lambda i: (0, 0)),
        ],
        out_specs=pl.BlockSpec((TC_BLOCK, HIDDEN), lambda i: (i, 0)),
        out_shape=jax.ShapeDtypeStruct((n, HIDDEN), jnp.float32),
    )(w_rows, cidx3, C_tab, W_pos_c, gamma, beta)


def kernel(word_x, age_x, seg_x, pos_x, W_word, W_age, W_seg, W_pos, gamma, beta):
    b, l = word_x.shape
    n = b * l
    word_i = word_x.reshape(n).astype(jnp.int32)
    age_i = age_x.reshape(n).astype(jnp.int32)
    seg_i = seg_x.reshape(n).astype(jnp.int32)
    pos_i = pos_x.reshape(n).astype(jnp.int32)
    cidx = (age_i + 128 * seg_i) * 256 + pos_i

    w_rows = _sc_gather_rows(W_word, word_i)

    W_age_p = jnp.zeros((128, HIDDEN), jnp.float32).at[:W_age.shape[0]].set(W_age)
    C_tab = jnp.concatenate(
        [W_age_p + W_seg[0], W_age_p + W_seg[1]], axis=0).astype(jnp.bfloat16)
    W_pos_c = W_pos[:POS_CLASSES].astype(jnp.bfloat16)
    out = _tc_sum_ln(w_rows, cidx, C_tab, W_pos_c,
                     gamma.reshape(1, HIDDEN), beta.reshape(1, HIDDEN))
    return out.reshape(b, l, HIDDEN)


# speedup vs baseline: 1.0970x; 1.0970x over previous
import jax
import jax.numpy as jnp
from jax.experimental import pallas as pl
from jax.experimental.pallas import tpu as pltpu
from jax.experimental.pallas import tpu_sc as plsc

HIDDEN = 128
EPS = 1e-5
GATHER_WINDOW = 256
TC_BLOCK = 1024
AS_CLASSES = 256
POS_CLASSES = 256
N_CHUNKS = 4


def _sc_gather_rows(table, flat_idx):
    n = flat_idx.shape[0]
    idx2 = flat_idx.reshape(1, n)
    mesh = plsc.VectorSubcoreMesh(core_axis_name="c", subcore_axis_name="s")

    @pl.kernel(
        out_type=jax.ShapeDtypeStruct((n, HIDDEN), table.dtype),
        mesh=mesh,
    )
    def gather_kernel(x_hbm, i_hbm, o_hbm):
        def body(i_vmem, o_vmem):
            pltpu.sync_copy(x_hbm.at[i_vmem.at[0]], o_vmem)

        pltpu.emit_pipeline(
            body,
            grid=(n // GATHER_WINDOW,),
            in_specs=[pl.BlockSpec((1, GATHER_WINDOW), index_map=lambda i: (0, i))],
            out_specs=[pl.BlockSpec((GATHER_WINDOW, HIDDEN), index_map=lambda i: (i, 0))],
            core_axis_name=("c", "s"),
            dimension_semantics=(pltpu.PARALLEL,),
        )(i_hbm, o_hbm)

    return gather_kernel(table, idx2)


def _tc_body(w_ref, cidx_ref, ctab_ref, wp_ref, g_ref, b_ref, o_ref):
    t = TC_BLOCK
    cvec = cidx_ref[0, 0, :]
    cb = jnp.broadcast_to(cvec[:, None], (t, AS_CLASSES))

    hi = cb >> 8
    as_oh = (hi == jax.lax.broadcasted_iota(jnp.int32, (t, AS_CLASSES), 1)
             ).astype(jnp.bfloat16)
    a_s = jnp.dot(as_oh, ctab_ref[...], preferred_element_type=jnp.float32)

    lo = cb & 255
    p_oh = (lo == jax.lax.broadcasted_iota(jnp.int32, (t, POS_CLASSES), 1)
            ).astype(jnp.bfloat16)
    p = jnp.dot(p_oh, wp_ref[...], preferred_element_type=jnp.float32)

    x = w_ref[...] + a_s + p

    xb = x.astype(jnp.bfloat16)
    j_mat = jnp.full((HIDDEN, HIDDEN), 1.0 / HIDDEN, dtype=jnp.bfloat16)
    mu = jnp.dot(xb, j_mat, preferred_element_type=jnp.float32)
    m2 = jnp.dot(xb * xb, j_mat, preferred_element_type=jnp.float32)
    var = m2 - mu * mu
    xn = (x - mu) * jax.lax.rsqrt(var + EPS)
    o_ref[...] = xn * g_ref[...] + b_ref[...]


def _tc_sum_ln_chunk(prev_out, w_chunk, cidx3, C_tab, W_pos_c, gamma, beta,
                     n_total, block_off):
    nbk = w_chunk.shape[0] // TC_BLOCK
    data_specs = [
        pl.BlockSpec((TC_BLOCK, HIDDEN), lambda i: (i, 0)),
        pl.BlockSpec((1, 1, TC_BLOCK), lambda i: (i, 0, 0)),
        pl.BlockSpec((AS_CLASSES, HIDDEN), lambda i: (0, 0)),
        pl.BlockSpec((POS_CLASSES, HIDDEN), lambda i: (0, 0)),
        pl.BlockSpec((1, HIDDEN), lambda i: (0, 0)),
        pl.BlockSpec((1, HIDDEN), lambda i: (0, 0)),
    ]
    data_args = (w_chunk, cidx3, C_tab, W_pos_c, gamma, beta)
    if prev_out is None:
        body, in_specs, args, aliases = _tc_body, data_specs, data_args, {}
    else:
        def body(prev_ref, *refs):
            del prev_ref
            _tc_body(*refs)
        in_specs = [pl.BlockSpec(memory_space=pl.ANY)] + data_specs
        args = (prev_out,) + data_args
        aliases = {0: 0}
    return pl.pallas_call(
        body,
        grid=(nbk,),
        in_specs=in_specs,
        out_specs=pl.BlockSpec((TC_BLOCK, HIDDEN),
                               lambda i: (i + block_off, 0)),
        out_shape=jax.ShapeDtypeStruct((n_total, HIDDEN), jnp.float32),
        input_output_aliases=aliases,
    )(*args)


def kernel(word_x, age_x, seg_x, pos_x, W_word, W_age, W_seg, W_pos, gamma, beta):
    b, l = word_x.shape
    n = b * l
    word_i = word_x.reshape(n).astype(jnp.int32)
    age_i = age_x.reshape(n).astype(jnp.int32)
    seg_i = seg_x.reshape(n).astype(jnp.int32)
    pos_i = pos_x.reshape(n).astype(jnp.int32)
    cidx = (age_i + 128 * seg_i) * 256 + pos_i

    W_age_p = jnp.zeros((128, HIDDEN), jnp.float32).at[:W_age.shape[0]].set(W_age)
    C_tab = jnp.concatenate(
        [W_age_p + W_seg[0], W_age_p + W_seg[1]], axis=0).astype(jnp.bfloat16)
    W_pos_c = W_pos[:POS_CLASSES].astype(jnp.bfloat16)
    g2 = gamma.reshape(1, HIDDEN)
    b2 = beta.reshape(1, HIDDEN)

    nk = n // N_CHUNKS
    nbk = nk // TC_BLOCK
    w_chunks = [_sc_gather_rows(W_word, word_i[k * nk:(k + 1) * nk])
                for k in range(N_CHUNKS)]
    cidx3 = cidx.reshape(n // TC_BLOCK, 1, TC_BLOCK)
    out = None
    for k in range(N_CHUNKS):
        out = _tc_sum_ln_chunk(out, w_chunks[k],
                               cidx3[k * nbk:(k + 1) * nbk],
                               C_tab, W_pos_c, g2, b2, n, k * nbk)
    return out.reshape(b, l, HIDDEN)


# speedup vs baseline: 1.3883x; 1.2656x over previous
import jax
import jax.numpy as jnp
from jax.experimental import pallas as pl
from jax.experimental.pallas import tpu as pltpu
from jax.experimental.pallas import tpu_sc as plsc

HIDDEN = 128
EPS = 1e-5
GATHER_WINDOW = 256
TC_BLOCK = 2048
AS_CLASSES = 256
POS_CLASSES = 256
CHUNK_SIZES = (16384, 61440, 63488, 63488)


def _sc_gather_rows(table, flat_idx):
    n = flat_idx.shape[0]
    idx2 = flat_idx.reshape(1, n)
    mesh = plsc.VectorSubcoreMesh(core_axis_name="c", subcore_axis_name="s")

    @pl.kernel(
        out_type=jax.ShapeDtypeStruct((n, HIDDEN), table.dtype),
        mesh=mesh,
    )
    def gather_kernel(x_hbm, i_hbm, o_hbm):
        def body(i_vmem, o_vmem):
            pltpu.sync_copy(x_hbm.at[i_vmem.at[0]], o_vmem)

        pltpu.emit_pipeline(
            body,
            grid=(n // GATHER_WINDOW,),
            in_specs=[pl.BlockSpec((1, GATHER_WINDOW), index_map=lambda i: (0, i))],
            out_specs=[pl.BlockSpec((GATHER_WINDOW, HIDDEN), index_map=lambda i: (i, 0))],
            core_axis_name=("c", "s"),
            dimension_semantics=(pltpu.PARALLEL,),
        )(i_hbm, o_hbm)

    return gather_kernel(table, idx2)


def _tc_body(w_ref, cidx_ref, ctab_ref, wp_ref, g_ref, b_ref, o_ref):
    t = TC_BLOCK
    cvec = cidx_ref[0, 0, :]
    cb = jnp.broadcast_to(cvec[:, None], (t, AS_CLASSES))

    hi = cb >> 8
    as_oh = (hi == jax.lax.broadcasted_iota(jnp.int32, (t, AS_CLASSES), 1)
             ).astype(jnp.bfloat16)
    a_s = jnp.dot(as_oh, ctab_ref[...], preferred_element_type=jnp.float32)

    lo = cb & 255
    p_oh = (lo == jax.lax.broadcasted_iota(jnp.int32, (t, POS_CLASSES), 1)
            ).astype(jnp.bfloat16)
    p = jnp.dot(p_oh, wp_ref[...], preferred_element_type=jnp.float32)

    x = w_ref[...] + a_s + p

    xb = x.astype(jnp.bfloat16)
    j_mat = jnp.full((HIDDEN, HIDDEN), 1.0 / HIDDEN, dtype=jnp.bfloat16)
    mu = jnp.dot(xb, j_mat, preferred_element_type=jnp.float32)
    m2 = jnp.dot(xb * xb, j_mat, preferred_element_type=jnp.float32)
    var = m2 - mu * mu
    xn = (x - mu) * jax.lax.rsqrt(var + EPS)
    o_ref[...] = xn * g_ref[...] + b_ref[...]


def _tc_sum_ln_chunk(prev_out, w_chunk, cidx3, C_tab, W_pos_c, gamma, beta,
                     n_total, block_off):
    nbk = w_chunk.shape[0] // TC_BLOCK
    data_specs = [
        pl.BlockSpec((TC_BLOCK, HIDDEN), lambda i: (i, 0)),
        pl.BlockSpec((1, 1, TC_BLOCK), lambda i: (i, 0, 0)),
        pl.BlockSpec((AS_CLASSES, HIDDEN), lambda i: (0, 0)),
        pl.BlockSpec((POS_CLASSES, HIDDEN), lambda i: (0, 0)),
        pl.BlockSpec((1, HIDDEN), lambda i: (0, 0)),
        pl.BlockSpec((1, HIDDEN), lambda i: (0, 0)),
    ]
    data_args = (w_chunk, cidx3, C_tab, W_pos_c, gamma, beta)
    if prev_out is None:
        body, in_specs, args, aliases = _tc_body, data_specs, data_args, {}
    else:
        def body(prev_ref, *refs):
            del prev_ref
            _tc_body(*refs)
        in_specs = [pl.BlockSpec(memory_space=pl.ANY)] + data_specs
        args = (prev_out,) + data_args
        aliases = {0: 0}
    return pl.pallas_call(
        body,
        grid=(nbk,),
        in_specs=in_specs,
        out_specs=pl.BlockSpec((TC_BLOCK, HIDDEN),
                               lambda i: (i + block_off, 0)),
        out_shape=jax.ShapeDtypeStruct((n_total, HIDDEN), jnp.float32),
        input_output_aliases=aliases,
    )(*args)


def kernel(word_x, age_x, seg_x, pos_x, W_word, W_age, W_seg, W_pos, gamma, beta):
    b, l = word_x.shape
    n = b * l
    word_i = word_x.reshape(n).astype(jnp.int32)
    age_i = age_x.reshape(n).astype(jnp.int32)
    seg_i = seg_x.reshape(n).astype(jnp.int32)
    pos_i = pos_x.reshape(n).astype(jnp.int32)
    cidx = (age_i + 128 * seg_i) * 256 + pos_i

    W_age_p = jnp.zeros((128, HIDDEN), jnp.float32).at[:W_age.shape[0]].set(W_age)
    C_tab = jnp.concatenate(
        [W_age_p + W_seg[0], W_age_p + W_seg[1]], axis=0).astype(jnp.bfloat16)
    W_pos_c = W_pos[:POS_CLASSES].astype(jnp.bfloat16)
    g2 = gamma.reshape(1, HIDDEN)
    b2 = beta.reshape(1, HIDDEN)

    offs = [0]
    for sz in CHUNK_SIZES:
        offs.append(offs[-1] + sz)
    w_chunks = [_sc_gather_rows(W_word, word_i[offs[k]:offs[k + 1]])
                for k in range(len(CHUNK_SIZES))]
    cidx3 = cidx.reshape(n // TC_BLOCK, 1, TC_BLOCK)
    out = None
    for k, sz in enumerate(CHUNK_SIZES):
        b0, b1 = offs[k] // TC_BLOCK, offs[k + 1] // TC_BLOCK
        out = _tc_sum_ln_chunk(out, w_chunks[k], cidx3[b0:b1],
                               C_tab, W_pos_c, g2, b2, n, b0)
    return out.reshape(b, l, HIDDEN)


# speedup vs baseline: 1.5450x; 1.1128x over previous
import jax
import jax.numpy as jnp
from jax.experimental import pallas as pl
from jax.experimental.pallas import tpu as pltpu
from jax.experimental.pallas import tpu_sc as plsc

HIDDEN = 128
EPS = 1e-5
GATHER_WINDOW = 256
TC_BLOCK = 4096
AS_CLASSES = 256
POS_CLASSES = 256
CHUNK_SIZES = (16384, 61440, 61440, 65536)


def _sc_gather_rows(table, flat_idx):
    n = flat_idx.shape[0]
    idx2 = flat_idx.reshape(1, n)
    mesh = plsc.VectorSubcoreMesh(core_axis_name="c", subcore_axis_name="s")

    @pl.kernel(
        out_type=jax.ShapeDtypeStruct((n, HIDDEN), table.dtype),
        mesh=mesh,
    )
    def gather_kernel(x_hbm, i_hbm, o_hbm):
        def body(i_vmem, o_vmem):
            pltpu.sync_copy(x_hbm.at[i_vmem.at[0]], o_vmem)

        pltpu.emit_pipeline(
            body,
            grid=(n // GATHER_WINDOW,),
            in_specs=[pl.BlockSpec((1, GATHER_WINDOW), index_map=lambda i: (0, i))],
            out_specs=[pl.BlockSpec((GATHER_WINDOW, HIDDEN), index_map=lambda i: (i, 0))],
            core_axis_name=("c", "s"),
            dimension_semantics=(pltpu.PARALLEL,),
        )(i_hbm, o_hbm)

    return gather_kernel(table, idx2)


def _tc_body(w_ref, cidx_ref, ctab_ref, wp_ref, g_ref, b_ref, o_ref):
    t = TC_BLOCK
    cvec = cidx_ref[0, 0, :]
    cb = jnp.broadcast_to(cvec[:, None], (t, AS_CLASSES))

    hi = cb >> 8
    as_oh = (hi == jax.lax.broadcasted_iota(jnp.int32, (t, AS_CLASSES), 1)
             ).astype(jnp.bfloat16)
    a_s = jnp.dot(as_oh, ctab_ref[...], preferred_element_type=jnp.float32)

    lo = cb & 255
    p_oh = (lo == jax.lax.broadcasted_iota(jnp.int32, (t, POS_CLASSES), 1)
            ).astype(jnp.bfloat16)
    p = jnp.dot(p_oh, wp_ref[...], preferred_element_type=jnp.float32)

    x = w_ref[...] + a_s + p

    xb = x.astype(jnp.bfloat16)
    j_mat = jnp.full((HIDDEN, HIDDEN), 1.0 / HIDDEN, dtype=jnp.bfloat16)
    mu = jnp.dot(xb, j_mat, preferred_element_type=jnp.float32)
    m2 = jnp.dot(xb * xb, j_mat, preferred_element_type=jnp.float32)
    var = m2 - mu * mu
    xn = (x - mu) * jax.lax.rsqrt(var + EPS)
    o_ref[...] = xn * g_ref[...] + b_ref[...]


def _tc_sum_ln_chunk(prev_out, w_chunk, cidx3, C_tab, W_pos_c, gamma, beta,
                     n_total, block_off):
    nbk = w_chunk.shape[0] // TC_BLOCK
    data_specs = [
        pl.BlockSpec((TC_BLOCK, HIDDEN), lambda i: (i, 0)),
        pl.BlockSpec((1, 1, TC_BLOCK), lambda i: (i, 0, 0)),
        pl.BlockSpec((AS_CLASSES, HIDDEN), lambda i: (0, 0)),
        pl.BlockSpec((POS_CLASSES, HIDDEN), lambda i: (0, 0)),
        pl.BlockSpec((1, HIDDEN), lambda i: (0, 0)),
        pl.BlockSpec((1, HIDDEN), lambda i: (0, 0)),
    ]
    data_args = (w_chunk, cidx3, C_tab, W_pos_c, gamma, beta)
    if prev_out is None:
        body, in_specs, args, aliases = _tc_body, data_specs, data_args, {}
    else:
        def body(prev_ref, *refs):
            del prev_ref
            _tc_body(*refs)
        in_specs = [pl.BlockSpec(memory_space=pl.ANY)] + data_specs
        args = (prev_out,) + data_args
        aliases = {0: 0}
    return pl.pallas_call(
        body,
        grid=(nbk,),
        in_specs=in_specs,
        out_specs=pl.BlockSpec((TC_BLOCK, HIDDEN),
                               lambda i: (i + block_off, 0)),
        out_shape=jax.ShapeDtypeStruct((n_total, HIDDEN), jnp.float32),
        input_output_aliases=aliases,
    )(*args)


def kernel(word_x, age_x, seg_x, pos_x, W_word, W_age, W_seg, W_pos, gamma, beta):
    b, l = word_x.shape
    n = b * l
    word_i = word_x.reshape(n).astype(jnp.int32)
    age_i = age_x.reshape(n).astype(jnp.int32)
    seg_i = seg_x.reshape(n).astype(jnp.int32)
    pos_i = pos_x.reshape(n).astype(jnp.int32)
    cidx = (age_i + 128 * seg_i) * 256 + pos_i

    W_age_p = jnp.zeros((128, HIDDEN), jnp.float32).at[:W_age.shape[0]].set(W_age)
    C_tab = jnp.concatenate(
        [W_age_p + W_seg[0], W_age_p + W_seg[1]], axis=0).astype(jnp.bfloat16)
    W_pos_c = W_pos[:POS_CLASSES].astype(jnp.bfloat16)
    g2 = gamma.reshape(1, HIDDEN)
    b2 = beta.reshape(1, HIDDEN)

    offs = [0]
    for sz in CHUNK_SIZES:
        offs.append(offs[-1] + sz)
    w_chunks = [_sc_gather_rows(W_word, word_i[offs[k]:offs[k + 1]])
                for k in range(len(CHUNK_SIZES))]
    cidx3 = cidx.reshape(n // TC_BLOCK, 1, TC_BLOCK)
    out = None
    for k, sz in enumerate(CHUNK_SIZES):
        b0, b1 = offs[k] // TC_BLOCK, offs[k + 1] // TC_BLOCK
        out = _tc_sum_ln_chunk(out, w_chunks[k], cidx3[b0:b1],
                               C_tab, W_pos_c, g2, b2, n, b0)
    return out.reshape(b, l, HIDDEN)


# speedup vs baseline: 1.6427x; 1.0633x over previous
import jax
import jax.numpy as jnp
from jax.experimental import pallas as pl
from jax.experimental.pallas import tpu as pltpu
from jax.experimental.pallas import tpu_sc as plsc

HIDDEN = 128
EPS = 1e-5
GATHER_WINDOW = 256
TC_BLOCK = 8192
AS_CLASSES = 256
POS_CLASSES = 256
CHUNK_SIZES = (16384, 57344, 65536, 65536)


def _sc_gather_rows(table, flat_idx):
    n = flat_idx.shape[0]
    idx2 = flat_idx.reshape(1, n)
    mesh = plsc.VectorSubcoreMesh(core_axis_name="c", subcore_axis_name="s")

    @pl.kernel(
        out_type=jax.ShapeDtypeStruct((n, HIDDEN), table.dtype),
        mesh=mesh,
    )
    def gather_kernel(x_hbm, i_hbm, o_hbm):
        def body(i_vmem, o_vmem):
            pltpu.sync_copy(x_hbm.at[i_vmem.at[0]], o_vmem)

        pltpu.emit_pipeline(
            body,
            grid=(n // GATHER_WINDOW,),
            in_specs=[pl.BlockSpec((1, GATHER_WINDOW), index_map=lambda i: (0, i))],
            out_specs=[pl.BlockSpec((GATHER_WINDOW, HIDDEN), index_map=lambda i: (i, 0))],
            core_axis_name=("c", "s"),
            dimension_semantics=(pltpu.PARALLEL,),
        )(i_hbm, o_hbm)

    return gather_kernel(table, idx2)


def _tc_body(w_ref, cidx_ref, ctab_ref, wp_ref, g_ref, b_ref, o_ref):
    t = TC_BLOCK
    cvec = cidx_ref[0, 0, :]
    cb = jnp.broadcast_to(cvec[:, None], (t, AS_CLASSES))

    hi = cb >> 8
    as_oh = (hi == jax.lax.broadcasted_iota(jnp.int32, (t, AS_CLASSES), 1)
             ).astype(jnp.bfloat16)
    a_s = jnp.dot(as_oh, ctab_ref[...], preferred_element_type=jnp.float32)

    lo = cb & 255
    p_oh = (lo == jax.lax.broadcasted_iota(jnp.int32, (t, POS_CLASSES), 1)
            ).astype(jnp.bfloat16)
    p = jnp.dot(p_oh, wp_ref[...], preferred_element_type=jnp.float32)

    x = w_ref[...] + a_s + p

    xb = x.astype(jnp.bfloat16)
    j_mat = jnp.full((HIDDEN, HIDDEN), 1.0 / HIDDEN, dtype=jnp.bfloat16)
    mu = jnp.dot(xb, j_mat, preferred_element_type=jnp.float32)
    m2 = jnp.dot(xb * xb, j_mat, preferred_element_type=jnp.float32)
    var = m2 - mu * mu
    xn = (x - mu) * jax.lax.rsqrt(var + EPS)
    o_ref[...] = xn * g_ref[...] + b_ref[...]


def _tc_sum_ln_chunk(prev_out, w_chunk, cidx3, C_tab, W_pos_c, gamma, beta,
                     n_total, block_off):
    nbk = w_chunk.shape[0] // TC_BLOCK
    data_specs = [
        pl.BlockSpec((TC_BLOCK, HIDDEN), lambda i: (i, 0)),
        pl.BlockSpec((1, 1, TC_BLOCK), lambda i: (i, 0, 0)),
        pl.BlockSpec((AS_CLASSES, HIDDEN), lambda i: (0, 0)),
        pl.BlockSpec((POS_CLASSES, HIDDEN), lambda i: (0, 0)),
        pl.BlockSpec((1, HIDDEN), lambda i: (0, 0)),
        pl.BlockSpec((1, HIDDEN), lambda i: (0, 0)),
    ]
    data_args = (w_chunk, cidx3, C_tab, W_pos_c, gamma, beta)
    if prev_out is None:
        body, in_specs, args, aliases = _tc_body, data_specs, data_args, {}
    else:
        def body(prev_ref, *refs):
            del prev_ref
            _tc_body(*refs)
        in_specs = [pl.BlockSpec(memory_space=pl.ANY)] + data_specs
        args = (prev_out,) + data_args
        aliases = {0: 0}
    return pl.pallas_call(
        body,
        grid=(nbk,),
        in_specs=in_specs,
        out_specs=pl.BlockSpec((TC_BLOCK, HIDDEN),
                               lambda i: (i + block_off, 0)),
        out_shape=jax.ShapeDtypeStruct((n_total, HIDDEN), jnp.float32),
        input_output_aliases=aliases,
    )(*args)


def kernel(word_x, age_x, seg_x, pos_x, W_word, W_age, W_seg, W_pos, gamma, beta):
    b, l = word_x.shape
    n = b * l
    word_i = word_x.reshape(n).astype(jnp.int32)
    age_i = age_x.reshape(n).astype(jnp.int32)
    seg_i = seg_x.reshape(n).astype(jnp.int32)
    pos_i = pos_x.reshape(n).astype(jnp.int32)
    cidx = (age_i + 128 * seg_i) * 256 + pos_i

    W_age_p = jnp.zeros((128, HIDDEN), jnp.float32).at[:W_age.shape[0]].set(W_age)
    C_tab = jnp.concatenate(
        [W_age_p + W_seg[0], W_age_p + W_seg[1]], axis=0).astype(jnp.bfloat16)
    W_pos_c = W_pos[:POS_CLASSES].astype(jnp.bfloat16)
    g2 = gamma.reshape(1, HIDDEN)
    b2 = beta.reshape(1, HIDDEN)

    offs = [0]
    for sz in CHUNK_SIZES:
        offs.append(offs[-1] + sz)
    w_chunks = [_sc_gather_rows(W_word, word_i[offs[k]:offs[k + 1]])
                for k in range(len(CHUNK_SIZES))]
    cidx3 = cidx.reshape(n // TC_BLOCK, 1, TC_BLOCK)
    out = None
    for k, sz in enumerate(CHUNK_SIZES):
        b0, b1 = offs[k] // TC_BLOCK, offs[k + 1] // TC_BLOCK
        out = _tc_sum_ln_chunk(out, w_chunks[k], cidx3[b0:b1],
                               C_tab, W_pos_c, g2, b2, n, b0)
    return out.reshape(b, l, HIDDEN)


# speedup vs baseline: 1.6607x; 1.0110x over previous
import jax
import jax.numpy as jnp
from jax.experimental import pallas as pl
from jax.experimental.pallas import tpu as pltpu
from jax.experimental.pallas import tpu_sc as plsc

HIDDEN = 128
EPS = 1e-5
GATHER_WINDOW = 256
TC_BLOCK = 8192
AS_CLASSES = 256
POS_CLASSES = 256
CHUNK_SIZES = (24576, 32768, 40960, 49152, 57344)


def _sc_gather_rows(table, idx2, off, nk):
    mesh = plsc.VectorSubcoreMesh(core_axis_name="c", subcore_axis_name="s")
    woff = off // GATHER_WINDOW

    @pl.kernel(
        out_type=jax.ShapeDtypeStruct((nk, HIDDEN), table.dtype),
        mesh=mesh,
    )
    def gather_kernel(x_hbm, i_hbm, o_hbm):
        def body(i_vmem, o_vmem):
            pltpu.sync_copy(x_hbm.at[i_vmem.at[0]], o_vmem)

        pltpu.emit_pipeline(
            body,
            grid=(nk // GATHER_WINDOW,),
            in_specs=[pl.BlockSpec((1, GATHER_WINDOW),
                                   index_map=lambda i: (0, i + woff))],
            out_specs=[pl.BlockSpec((GATHER_WINDOW, HIDDEN), index_map=lambda i: (i, 0))],
            core_axis_name=("c", "s"),
            dimension_semantics=(pltpu.PARALLEL,),
        )(i_hbm, o_hbm)

    return gather_kernel(table, idx2)


def _tc_body(w_ref, cidx_ref, ctab_ref, wp_ref, g_ref, b_ref, o_ref):
    t = TC_BLOCK
    cvec = cidx_ref[0, 0, :]
    cb = jnp.broadcast_to(cvec[:, None], (t, AS_CLASSES))

    hi = cb >> 8
    as_oh = (hi == jax.lax.broadcasted_iota(jnp.int32, (t, AS_CLASSES), 1)
             ).astype(jnp.bfloat16)
    a_s = jnp.dot(as_oh, ctab_ref[...], preferred_element_type=jnp.float32)

    lo = cb & 255
    p_oh = (lo == jax.lax.broadcasted_iota(jnp.int32, (t, POS_CLASSES), 1)
            ).astype(jnp.bfloat16)
    p = jnp.dot(p_oh, wp_ref[...], preferred_element_type=jnp.float32)

    x = w_ref[...] + a_s + p

    xb = x.astype(jnp.bfloat16)
    j_mat = jnp.full((HIDDEN, HIDDEN), 1.0 / HIDDEN, dtype=jnp.bfloat16)
    mu = jnp.dot(xb, j_mat, preferred_element_type=jnp.float32)
    m2 = jnp.dot(xb * xb, j_mat, preferred_element_type=jnp.float32)
    var = m2 - mu * mu
    xn = (x - mu) * jax.lax.rsqrt(var + EPS)
    o_ref[...] = xn * g_ref[...] + b_ref[...]


def _tc_sum_ln_chunk(prev_out, w_chunk, cidx3, C_tab, W_pos_c, gamma, beta,
                     n_total, block_off):
    nbk = w_chunk.shape[0] // TC_BLOCK
    data_specs = [
        pl.BlockSpec((TC_BLOCK, HIDDEN), lambda i: (i, 0)),
        pl.BlockSpec((1, 1, TC_BLOCK), lambda i: (i + block_off, 0, 0)),
        pl.BlockSpec((AS_CLASSES, HIDDEN), lambda i: (0, 0)),
        pl.BlockSpec((POS_CLASSES, HIDDEN), lambda i: (0, 0)),
        pl.BlockSpec((1, HIDDEN), lambda i: (0, 0)),
        pl.BlockSpec((1, HIDDEN), lambda i: (0, 0)),
    ]
    data_args = (w_chunk, cidx3, C_tab, W_pos_c, gamma, beta)
    if prev_out is None:
        body, in_specs, args, aliases = _tc_body, data_specs, data_args, {}
    else:
        def body(prev_ref, *refs):
            del prev_ref
            _tc_body(*refs)
        in_specs = [pl.BlockSpec(memory_space=pl.ANY)] + data_specs
        args = (prev_out,) + data_args
        aliases = {0: 0}
    return pl.pallas_call(
        body,
        grid=(nbk,),
        in_specs=in_specs,
        out_specs=pl.BlockSpec((TC_BLOCK, HIDDEN),
                               lambda i: (i + block_off, 0)),
        out_shape=jax.ShapeDtypeStruct((n_total, HIDDEN), jnp.float32),
        input_output_aliases=aliases,
    )(*args)


def kernel(word_x, age_x, seg_x, pos_x, W_word, W_age, W_seg, W_pos, gamma, beta):
    b, l = word_x.shape
    n = b * l
    word_i = word_x.reshape(n).astype(jnp.int32)
    age_i = age_x.reshape(n).astype(jnp.int32)
    seg_i = seg_x.reshape(n).astype(jnp.int32)
    pos_i = pos_x.reshape(n).astype(jnp.int32)
    cidx = (age_i + 128 * seg_i) * 256 + pos_i

    W_age_p = jnp.zeros((128, HIDDEN), jnp.float32).at[:W_age.shape[0]].set(W_age)
    C_tab = jnp.concatenate(
        [W_age_p + W_seg[0], W_age_p + W_seg[1]], axis=0).astype(jnp.bfloat16)
    W_pos_c = W_pos[:POS_CLASSES].astype(jnp.bfloat16)
    g2 = gamma.reshape(1, HIDDEN)
    b2 = beta.reshape(1, HIDDEN)

    offs = [0]
    for sz in CHUNK_SIZES:
        offs.append(offs[-1] + sz)
    word_i2 = word_i.reshape(1, n)
    w_chunks = [_sc_gather_rows(W_word, word_i2, offs[k], sz)
                for k, sz in enumerate(CHUNK_SIZES)]
    cidx3 = cidx.reshape(n // TC_BLOCK, 1, TC_BLOCK)
    out = None
    for k, sz in enumerate(CHUNK_SIZES):
        out = _tc_sum_ln_chunk(out, w_chunks[k], cidx3,
                               C_tab, W_pos_c, g2, b2, n,
                               offs[k] // TC_BLOCK)
    return out.reshape(b, l, HIDDEN)


# speedup vs baseline: 1.7389x; 1.0470x over previous
import jax
import jax.numpy as jnp
from jax.experimental import pallas as pl
from jax.experimental.pallas import tpu as pltpu
from jax.experimental.pallas import tpu_sc as plsc

HIDDEN = 128
EPS = 1e-5
GATHER_WINDOW = 256
TC_BLOCK = 8192
AS_CLASSES = 256
POS_CLASSES = 256
CHUNK_SIZES = (24576, 32768, 40960, 49152, 57344)


def _sc_gather_rows(table, idx2, off, nk):
    mesh = plsc.VectorSubcoreMesh(core_axis_name="c", subcore_axis_name="s")
    woff = off // GATHER_WINDOW

    @pl.kernel(
        out_type=jax.ShapeDtypeStruct((nk, HIDDEN), table.dtype),
        mesh=mesh,
    )
    def gather_kernel(x_hbm, i_hbm, o_hbm):
        def body(i_vmem, o_vmem):
            pltpu.sync_copy(x_hbm.at[i_vmem.at[0]], o_vmem)

        pltpu.emit_pipeline(
            body,
            grid=(nk // GATHER_WINDOW,),
            in_specs=[pl.BlockSpec((1, GATHER_WINDOW),
                                   index_map=lambda i: (0, i + woff))],
            out_specs=[pl.BlockSpec((GATHER_WINDOW, HIDDEN), index_map=lambda i: (i, 0))],
            core_axis_name=("c", "s"),
            dimension_semantics=(pltpu.PARALLEL,),
        )(i_hbm, o_hbm)

    return gather_kernel(table, idx2)


def _tc_body(w_ref, cidx_ref, ctab_ref, wp_ref, jj_ref, g_ref, b_ref, o_ref):
    t = TC_BLOCK
    cvec = cidx_ref[0, 0, :]
    cb = jnp.broadcast_to(cvec[:, None], (t, AS_CLASSES))

    hi = cb >> 8
    as_oh = (hi == jax.lax.broadcasted_iota(jnp.int32, (t, AS_CLASSES), 1)
             ).astype(jnp.bfloat16)
    a_s = jnp.dot(as_oh, ctab_ref[...], preferred_element_type=jnp.float32)

    lo = cb & 255
    p_oh = (lo == jax.lax.broadcasted_iota(jnp.int32, (t, POS_CLASSES), 1)
            ).astype(jnp.bfloat16)
    p = jnp.dot(p_oh, wp_ref[...], preferred_element_type=jnp.float32)

    x = w_ref[...] + a_s + p

    xb = x.astype(jnp.bfloat16)
    packed = jnp.concatenate([xb, xb * xb], axis=1)
    fused = jnp.dot(packed, jj_ref[...], preferred_element_type=jnp.float32)
    mu = fused[:, :HIDDEN]
    var = fused[:, HIDDEN:] - mu * mu
    xn = (x - mu) * jax.lax.rsqrt(var + EPS)
    o_ref[...] = xn * g_ref[...] + b_ref[...]


def _tc_sum_ln_chunk(prev_out, w_chunk, cidx3, C_tab, W_pos_c, JJ, gamma,
                     beta, n_total, block_off):
    nbk = w_chunk.shape[0] // TC_BLOCK
    data_specs = [
        pl.BlockSpec((TC_BLOCK, HIDDEN), lambda i: (i, 0)),
        pl.BlockSpec((1, 1, TC_BLOCK), lambda i: (i + block_off, 0, 0)),
        pl.BlockSpec((AS_CLASSES, HIDDEN), lambda i: (0, 0)),
        pl.BlockSpec((POS_CLASSES, HIDDEN), lambda i: (0, 0)),
        pl.BlockSpec((2 * HIDDEN, 2 * HIDDEN), lambda i: (0, 0)),
        pl.BlockSpec((1, HIDDEN), lambda i: (0, 0)),
        pl.BlockSpec((1, HIDDEN), lambda i: (0, 0)),
    ]
    data_args = (w_chunk, cidx3, C_tab, W_pos_c, JJ, gamma, beta)
    if prev_out is None:
        body, in_specs, args, aliases = _tc_body, data_specs, data_args, {}
    else:
        def body(prev_ref, *refs):
            del prev_ref
            _tc_body(*refs)
        in_specs = [pl.BlockSpec(memory_space=pl.ANY)] + data_specs
        args = (prev_out,) + data_args
        aliases = {0: 0}
    return pl.pallas_call(
        body,
        grid=(nbk,),
        in_specs=in_specs,
        out_specs=pl.BlockSpec((TC_BLOCK, HIDDEN),
                               lambda i: (i + block_off, 0)),
        out_shape=jax.ShapeDtypeStruct((n_total, HIDDEN), jnp.float32),
        input_output_aliases=aliases,
    )(*args)


def kernel(word_x, age_x, seg_x, pos_x, W_word, W_age, W_seg, W_pos, gamma, beta):
    b, l = word_x.shape
    n = b * l
    word_i = word_x.reshape(n).astype(jnp.int32)
    age_i = age_x.reshape(n).astype(jnp.int32)
    seg_i = seg_x.reshape(n).astype(jnp.int32)
    pos_i = pos_x.reshape(n).astype(jnp.int32)
    cidx = (age_i + 128 * seg_i) * 256 + pos_i

    W_age_p = jnp.zeros((128, HIDDEN), jnp.float32).at[:W_age.shape[0]].set(W_age)
    C_tab = jnp.concatenate(
        [W_age_p + W_seg[0], W_age_p + W_seg[1]], axis=0).astype(jnp.bfloat16)
    W_pos_c = W_pos[:POS_CLASSES].astype(jnp.bfloat16)
    j_blk = jnp.full((HIDDEN, HIDDEN), 1.0 / HIDDEN, jnp.float32)
    z_blk = jnp.zeros((HIDDEN, HIDDEN), jnp.float32)
    JJ = jnp.block([[j_blk, z_blk], [z_blk, j_blk]]).astype(jnp.bfloat16)
    g2 = gamma.reshape(1, HIDDEN)
    b2 = beta.reshape(1, HIDDEN)

    offs = [0]
    for sz in CHUNK_SIZES:
        offs.append(offs[-1] + sz)
    word_i2 = word_i.reshape(1, n)
    w_chunks = [_sc_gather_rows(W_word, word_i2, offs[k], sz)
                for k, sz in enumerate(CHUNK_SIZES)]
    cidx3 = cidx.reshape(n // TC_BLOCK, 1, TC_BLOCK)
    out = None
    for k, sz in enumerate(CHUNK_SIZES):
        out = _tc_sum_ln_chunk(out, w_chunks[k], cidx3,
                               C_tab, W_pos_c, JJ, g2, b2, n,
                               offs[k] // TC_BLOCK)
    return out.reshape(b, l, HIDDEN)
